# Initial kernel scaffold; baseline (speedup 1.0000x reference)
import jax
import jax.numpy as jnp
from jax.experimental import pallas as pl


def kernel(x_stroke, edge_index_intersects, edge_index_temp_previous, edge_index_represented_by, W_msg_head, W_self_head, b_head, W_msg, W_self, b):
    raise NotImplementedError("write your pallas kernel here")


# speedup vs baseline: 10.3776x; 10.3776x over previous
import functools

import jax
import jax.numpy as jnp
from jax import lax
from jax.experimental import pallas as pl
from jax.experimental.pallas import tpu as pltpu
from jax.experimental.pallas import tpu_sc as plsc

N = 100000
NP = 100096
E = 1600000
HID = 32
HALF = 16
CHUNK = 128
NCH = 12544
PAD = NCH * CHUNK - E
NT = 16
CPT = NCH // NT
GRP = 8
NGRP = CPT // GRP
STRIPE = NP // NT
ZROWS = 368
NZC = STRIPE // ZROWS
DUMP = N

_mesh = plsc.VectorSubcoreMesh(core_axis_name="c", subcore_axis_name="s",
                               num_cores=2, num_subcores=NT)


def _counts_body(dst_hbm, ones_hbm, cnt_hbm, acc, dstb, ones_v, ssem):
    cid = lax.axis_index("c")
    tid = lax.axis_index("s")
    pltpu.sync_copy(ones_hbm, ones_v)
    rel = cid * 2

    @pl.loop(0, NZC)
    def _zero(k):
        pltpu.sync_copy(ones_v.at[pl.ds(CHUNK, ZROWS)],
                        acc.at[pl.ds(tid * STRIPE + k * ZROWS, ZROWS)])

    plsc.subcore_barrier()

    @pl.loop(0, NGRP)
    def _grp(g):
        base = tid * CPT + g * GRP
        pltpu.sync_copy(dst_hbm.at[rel, pl.ds(base, GRP)], dstb)
        cps = [pltpu.async_copy(ones_v.at[pl.ds(0, CHUNK)],
                                acc.at[dstb.at[j]], ssem, add=True)
               for j in range(GRP)]
        for cp in cps:
            cp.wait()

    plsc.subcore_barrier()
    pltpu.sync_copy(acc.at[pl.ds(tid * STRIPE, STRIPE)],
                    cnt_hbm.at[cid, pl.ds(tid * STRIPE, STRIPE)])


def _spmm_body(tbl_hbm, src_hbm, dst_hbm, ones_hbm, agg_hbm,
               acc, srcb, dstb, rows, zed, gsem, ssem):
    cid = lax.axis_index("c")
    tid = lax.axis_index("s")
    pltpu.sync_copy(ones_hbm.at[pl.ds(CHUNK, ZROWS)], zed)

    for r in range(3):
        @pl.loop(0, NZC)
        def _zero(k):
            pltpu.sync_copy(zed, acc.at[pl.ds(tid * STRIPE + k * ZROWS, ZROWS)])

        plsc.subcore_barrier()

        @pl.loop(0, NGRP)
        def _grp(g):
            base = tid * CPT + g * GRP
            pltpu.sync_copy(src_hbm.at[r, cid, pl.ds(base, GRP)], srcb)
            pltpu.sync_copy(dst_hbm.at[r, pl.ds(base, GRP)], dstb)
            cps = [pltpu.async_copy(tbl_hbm.at[srcb.at[j]], rows.at[j], gsem)
                   for j in range(GRP)]
            for cp in cps:
                cp.wait()
            cps2 = [pltpu.async_copy(rows.at[j], acc.at[dstb.at[j]], ssem,
                                     add=True)
                    for j in range(GRP)]
            for cp in cps2:
                cp.wait()

        plsc.subcore_barrier()
        pltpu.sync_copy(acc.at[pl.ds(tid * STRIPE, STRIPE)],
                        agg_hbm.at[r, cid, pl.ds(tid * STRIPE, STRIPE)])
        plsc.subcore_barrier()


_counts_call = pl.kernel(
    _counts_body,
    out_type=jax.ShapeDtypeStruct((2, NP, HALF), jnp.float32),
    mesh=_mesh,
    compiler_params=pltpu.CompilerParams(use_tc_tiling_on_sc=False),
    scratch_types=[
        pltpu.VMEM_SHARED((NP, HALF), jnp.float32),
        pltpu.VMEM((GRP, CHUNK), jnp.int32),
        pltpu.VMEM((CHUNK + ZROWS, HALF), jnp.float32),
        pltpu.SemaphoreType.DMA,
    ],
    name="sc_counts",
)

_spmm_call = pl.kernel(
    _spmm_body,
    out_type=jax.ShapeDtypeStruct((3, 2, NP, HALF), jnp.float32),
    mesh=_mesh,
    compiler_params=pltpu.CompilerParams(use_tc_tiling_on_sc=False),
    scratch_types=[
        pltpu.VMEM_SHARED((NP, HALF), jnp.float32),
        pltpu.VMEM((GRP, CHUNK), jnp.int32),
        pltpu.VMEM((GRP, CHUNK), jnp.int32),
        pltpu.VMEM((GRP, CHUNK, HALF), jnp.float32),
        pltpu.VMEM((ZROWS, HALF), jnp.float32),
        pltpu.SemaphoreType.DMA,
        pltpu.SemaphoreType.DMA,
    ],
    name="sc_spmm",
)


def _tc_body(agg_ref, h_ref, cnt_ref, m_ref, b_ref, out_ref, *, mode):
    inv0 = 1.0 / jnp.maximum(cnt_ref[0], 1.0)
    inv2 = 1.0 / jnp.maximum(cnt_ref[1], 1.0)
    z = jnp.concatenate([
        agg_ref[0, 0] * inv0, agg_ref[0, 1] * inv0,
        agg_ref[1, 0], agg_ref[1, 1],
        agg_ref[2, 0] * inv2, agg_ref[2, 1] * inv2,
        h_ref[0], h_ref[1],
    ], axis=-1)
    o = jnp.dot(z, m_ref[...], preferred_element_type=jnp.float32)
    o = o + b_ref[...]
    if mode == "head":
        hn = o
    else:
        h_full = jnp.concatenate([h_ref[0], h_ref[1]], axis=-1)
        hn = h_full + jnp.maximum(o, 0.0)
    if mode == "final":
        out_ref[...] = jnp.maximum(hn, 0.0)
    else:
        out_ref[0] = hn[:, :HALF]
        out_ref[1] = hn[:, HALF:]


_TCB = 1088


def _tc_call(agg, h_split, cnt, m, bias, mode):
    grid = (NP // _TCB,)
    in_specs = [
        pl.BlockSpec((3, 2, _TCB, HALF), lambda i: (0, 0, i, 0)),
        pl.BlockSpec((2, _TCB, HALF), lambda i: (0, i, 0)),
        pl.BlockSpec((2, _TCB, HALF), lambda i: (0, i, 0)),
        pl.BlockSpec((4 * HID, HID), lambda i: (0, 0)),
        pl.BlockSpec((1, HID), lambda i: (0, 0)),
    ]
    if mode == "final":
        out_spec = pl.BlockSpec((_TCB, HID), lambda i: (i, 0))
        out_shape = jax.ShapeDtypeStruct((NP, HID), jnp.float32)
    else:
        out_spec = pl.BlockSpec((2, _TCB, HALF), lambda i: (0, i, 0))
        out_shape = jax.ShapeDtypeStruct((2, NP, HALF), jnp.float32)
    return pl.pallas_call(
        functools.partial(_tc_body, mode=mode),
        grid=grid,
        in_specs=in_specs,
        out_specs=out_spec,
        out_shape=out_shape,
        name=f"tc_dense_{mode}",
    )(agg, h_split, cnt, m, bias)


def _prep_edges(ei):
    src = ei[0].astype(jnp.int32)
    dst = ei[1].astype(jnp.int32)
    src = jnp.concatenate([src, jnp.zeros((PAD,), jnp.int32)])
    dst = jnp.concatenate([dst, jnp.full((PAD,), DUMP, jnp.int32)])
    src2 = jnp.stack([src, src + NP]).reshape(2, NCH, CHUNK)
    return src2, dst.reshape(NCH, CHUNK)


def _pad_rows(w):
    return jnp.pad(w, ((0, HID - w.shape[0]), (0, 0)))


def kernel(x_stroke, edge_index_intersects, edge_index_temp_previous,
           edge_index_represented_by, W_msg_head, W_self_head, b_head,
           W_msg, W_self, b):
    f32 = jnp.float32
    prepped = [_prep_edges(e) for e in (edge_index_intersects,
                                        edge_index_temp_previous,
                                        edge_index_represented_by)]
    src_all = jnp.stack([p[0] for p in prepped])
    dst_all = jnp.stack([p[1] for p in prepped])

    ones_z = jnp.concatenate([jnp.ones((CHUNK, HALF), f32),
                              jnp.zeros((ZROWS, HALF), f32)])

    m_head = jnp.concatenate([_pad_rows(W_msg_head[0].astype(f32)),
                              _pad_rows(W_msg_head[1].astype(f32)),
                              _pad_rows(W_msg_head[2].astype(f32)),
                              _pad_rows(W_self_head.astype(f32).sum(0))])
    b_hd = b_head.astype(f32).sum(0).reshape(1, HID)
    ms = [jnp.concatenate([W_msg[l, 0], W_msg[l, 1], W_msg[l, 2],
                           W_self[l].sum(0)]).astype(f32)
          for l in range(4)]
    bs = [b[l].astype(f32).sum(0).reshape(1, HID) for l in range(4)]

    x32 = jnp.pad(x_stroke.astype(f32),
                  ((0, NP - N), (0, HID - x_stroke.shape[1])))
    h_split = jnp.transpose(x32.reshape(NP, 2, HALF), (1, 0, 2))

    cnt = _counts_call(dst_all, ones_z)

    agg = _spmm_call(h_split.reshape(2 * NP, HALF), src_all, dst_all, ones_z)
    h_split = _tc_call(agg, h_split, cnt, m_head, b_hd, "head")
    for l in range(4):
        agg = _spmm_call(h_split.reshape(2 * NP, HALF), src_all, dst_all,
                         ones_z)
        mode = "final" if l == 3 else "mid"
        h_split = _tc_call(agg, h_split, cnt, ms[l], bs[l], mode)
    return h_split[:N]


# speedup vs baseline: 12.5187x; 1.2063x over previous
import functools

import jax
import jax.numpy as jnp
from jax import lax
from jax.experimental import pallas as pl
from jax.experimental.pallas import tpu as pltpu
from jax.experimental.pallas import tpu_sc as plsc

N = 100000
NP = 100096
E = 1600000
HID = 32
HALF = 16
CHUNK = 128
NCH = 12544
PAD = NCH * CHUNK - E
NT = 16
CPT = NCH // NT
GRP = 8
NGRP = CPT // GRP
STRIPE = NP // NT
ZROWS = 368
NZC = STRIPE // ZROWS
DUMP = N

_mesh = plsc.VectorSubcoreMesh(core_axis_name="c", subcore_axis_name="s",
                               num_cores=2, num_subcores=NT)


def _counts_body(dst_hbm, ones_hbm, cnt_hbm, acc, dstb, ones_v, ssem):
    cid = lax.axis_index("c")
    tid = lax.axis_index("s")
    pltpu.sync_copy(ones_hbm, ones_v)
    rel = cid * 2

    @pl.loop(0, NZC)
    def _zero(k):
        pltpu.sync_copy(ones_v.at[pl.ds(CHUNK, ZROWS)],
                        acc.at[pl.ds(tid * STRIPE + k * ZROWS, ZROWS)])

    plsc.subcore_barrier()

    @pl.loop(0, NGRP)
    def _grp(g):
        base = tid * CPT + g * GRP
        pltpu.sync_copy(dst_hbm.at[rel, pl.ds(base, GRP)], dstb)
        cps = [pltpu.async_copy(ones_v.at[pl.ds(0, CHUNK)],
                                acc.at[dstb.at[j]], ssem, add=True)
               for j in range(GRP)]
        for cp in cps:
            cp.wait()

    plsc.subcore_barrier()
    pltpu.sync_copy(acc.at[pl.ds(tid * STRIPE, STRIPE)],
                    cnt_hbm.at[cid, pl.ds(tid * STRIPE, STRIPE)])


def _spmm_body(tbl_hbm, src_hbm, dst_hbm, ones_hbm, agg_hbm,
               acc, srcb, dstb, rows, zed, gsem, ssems, isem):
    cid = lax.axis_index("c")
    tid = lax.axis_index("s")
    pltpu.sync_copy(ones_hbm.at[pl.ds(CHUNK, ZROWS)], zed)

    def fire_idx(r, g, p):
        base = tid * CPT + g * GRP
        pltpu.async_copy(src_hbm.at[r, cid, pl.ds(base, GRP)], srcb.at[p],
                         isem)
        pltpu.async_copy(dst_hbm.at[r, pl.ds(base, GRP)], dstb.at[p], isem)

    def wait_idx(r, g, p):
        base = tid * CPT + g * GRP
        pltpu.make_async_copy(src_hbm.at[r, cid, pl.ds(base, GRP)],
                              srcb.at[p], isem).wait()
        pltpu.make_async_copy(dst_hbm.at[r, pl.ds(base, GRP)], dstb.at[p],
                              isem).wait()

    for r in range(3):
        @pl.loop(0, NZC)
        def _zero(k):
            pltpu.sync_copy(zed, acc.at[pl.ds(tid * STRIPE + k * ZROWS, ZROWS)])

        plsc.subcore_barrier()
        fire_idx(r, 0, 0)

        @pl.loop(0, NGRP)
        def _grp(g):
            p = lax.rem(g, 2)
            wait_idx(r, g, p)

            @pl.when(g > 0)
            def _drain_prev():
                for j in range(GRP):
                    pltpu.make_async_copy(rows.at[j], acc.at[dstb.at[p, j]],
                                          ssems.at[j]).wait()

            @pl.when(g < NGRP - 1)
            def _pf():
                fire_idx(r, g + 1, 1 - p)

            cps = [pltpu.async_copy(tbl_hbm.at[srcb.at[p, j]], rows.at[j],
                                    gsem)
                   for j in range(GRP)]
            for cp in cps:
                cp.wait()
            for j in range(GRP):
                pltpu.async_copy(rows.at[j], acc.at[dstb.at[p, j]],
                                 ssems.at[j], add=True)

        pfin = lax.rem(NGRP - 1, 2)
        for j in range(GRP):
            pltpu.make_async_copy(rows.at[j], acc.at[dstb.at[pfin, j]],
                                  ssems.at[j]).wait()

        plsc.subcore_barrier()
        pltpu.sync_copy(acc.at[pl.ds(tid * STRIPE, STRIPE)],
                        agg_hbm.at[r, cid, pl.ds(tid * STRIPE, STRIPE)])
        plsc.subcore_barrier()


_counts_call = pl.kernel(
    _counts_body,
    out_type=jax.ShapeDtypeStruct((2, NP, HALF), jnp.float32),
    mesh=_mesh,
    compiler_params=pltpu.CompilerParams(use_tc_tiling_on_sc=False),
    scratch_types=[
        pltpu.VMEM_SHARED((NP, HALF), jnp.float32),
        pltpu.VMEM((GRP, CHUNK), jnp.int32),
        pltpu.VMEM((CHUNK + ZROWS, HALF), jnp.float32),
        pltpu.SemaphoreType.DMA,
    ],
    name="sc_counts",
)

_spmm_call = pl.kernel(
    _spmm_body,
    out_type=jax.ShapeDtypeStruct((3, 2, NP, HALF), jnp.float32),
    mesh=_mesh,
    compiler_params=pltpu.CompilerParams(use_tc_tiling_on_sc=False),
    scratch_types=[
        pltpu.VMEM_SHARED((NP, HALF), jnp.float32),
        pltpu.VMEM((2, GRP, CHUNK), jnp.int32),
        pltpu.VMEM((2, GRP, CHUNK), jnp.int32),
        pltpu.VMEM((GRP, CHUNK, HALF), jnp.float32),
        pltpu.VMEM((ZROWS, HALF), jnp.float32),
        pltpu.SemaphoreType.DMA,
        pltpu.SemaphoreType.DMA((GRP,)),
        pltpu.SemaphoreType.DMA,
    ],
    name="sc_spmm",
)


def _tc_body(agg_ref, h_ref, cnt_ref, m_ref, b_ref, out_ref, *, mode):
    inv0 = 1.0 / jnp.maximum(cnt_ref[0], 1.0)
    inv2 = 1.0 / jnp.maximum(cnt_ref[1], 1.0)
    z = jnp.concatenate([
        agg_ref[0, 0] * inv0, agg_ref[0, 1] * inv0,
        agg_ref[1, 0], agg_ref[1, 1],
        agg_ref[2, 0] * inv2, agg_ref[2, 1] * inv2,
        h_ref[0], h_ref[1],
    ], axis=-1)
    o = jnp.dot(z, m_ref[...], preferred_element_type=jnp.float32)
    o = o + b_ref[...]
    if mode == "head":
        hn = o
    else:
        h_full = jnp.concatenate([h_ref[0], h_ref[1]], axis=-1)
        hn = h_full + jnp.maximum(o, 0.0)
    if mode == "final":
        out_ref[...] = jnp.maximum(hn, 0.0)
    else:
        out_ref[0] = hn[:, :HALF]
        out_ref[1] = hn[:, HALF:]


_TCB = 1088


def _tc_call(agg, h_split, cnt, m, bias, mode):
    grid = (NP // _TCB,)
    in_specs = [
        pl.BlockSpec((3, 2, _TCB, HALF), lambda i: (0, 0, i, 0)),
        pl.BlockSpec((2, _TCB, HALF), lambda i: (0, i, 0)),
        pl.BlockSpec((2, _TCB, HALF), lambda i: (0, i, 0)),
        pl.BlockSpec((4 * HID, HID), lambda i: (0, 0)),
        pl.BlockSpec((1, HID), lambda i: (0, 0)),
    ]
    if mode == "final":
        out_spec = pl.BlockSpec((_TCB, HID), lambda i: (i, 0))
        out_shape = jax.ShapeDtypeStruct((NP, HID), jnp.float32)
    else:
        out_spec = pl.BlockSpec((2, _TCB, HALF), lambda i: (0, i, 0))
        out_shape = jax.ShapeDtypeStruct((2, NP, HALF), jnp.float32)
    return pl.pallas_call(
        functools.partial(_tc_body, mode=mode),
        grid=grid,
        in_specs=in_specs,
        out_specs=out_spec,
        out_shape=out_shape,
        name=f"tc_dense_{mode}",
    )(agg, h_split, cnt, m, bias)


def _prep_edges(ei):
    src = ei[0].astype(jnp.int32)
    dst = ei[1].astype(jnp.int32)
    src = jnp.concatenate([src, jnp.zeros((PAD,), jnp.int32)])
    dst = jnp.concatenate([dst, jnp.full((PAD,), DUMP, jnp.int32)])
    src2 = jnp.stack([src, src + NP]).reshape(2, NCH, CHUNK)
    return src2, dst.reshape(NCH, CHUNK)


def _pad_rows(w):
    return jnp.pad(w, ((0, HID - w.shape[0]), (0, 0)))


def kernel(x_stroke, edge_index_intersects, edge_index_temp_previous,
           edge_index_represented_by, W_msg_head, W_self_head, b_head,
           W_msg, W_self, b):
    f32 = jnp.float32
    prepped = [_prep_edges(e) for e in (edge_index_intersects,
                                        edge_index_temp_previous,
                                        edge_index_represented_by)]
    src_all = jnp.stack([p[0] for p in prepped])
    dst_all = jnp.stack([p[1] for p in prepped])

    ones_z = jnp.concatenate([jnp.ones((CHUNK, HALF), f32),
                              jnp.zeros((ZROWS, HALF), f32)])

    m_head = jnp.concatenate([_pad_rows(W_msg_head[0].astype(f32)),
                              _pad_rows(W_msg_head[1].astype(f32)),
                              _pad_rows(W_msg_head[2].astype(f32)),
                              _pad_rows(W_self_head.astype(f32).sum(0))])
    b_hd = b_head.astype(f32).sum(0).reshape(1, HID)
    ms = [jnp.concatenate([W_msg[l, 0], W_msg[l, 1], W_msg[l, 2],
                           W_self[l].sum(0)]).astype(f32)
          for l in range(4)]
    bs = [b[l].astype(f32).sum(0).reshape(1, HID) for l in range(4)]

    x32 = jnp.pad(x_stroke.astype(f32),
                  ((0, NP - N), (0, HID - x_stroke.shape[1])))
    h_split = jnp.transpose(x32.reshape(NP, 2, HALF), (1, 0, 2))

    cnt = _counts_call(dst_all, ones_z)

    agg = _spmm_call(h_split.reshape(2 * NP, HALF), src_all, dst_all, ones_z)
    h_split = _tc_call(agg, h_split, cnt, m_head, b_hd, "head")
    for l in range(4):
        agg = _spmm_call(h_split.reshape(2 * NP, HALF), src_all, dst_all,
                         ones_z)
        mode = "final" if l == 3 else "mid"
        h_split = _tc_call(agg, h_split, cnt, ms[l], bs[l], mode)
    return h_split[:N]


# speedup vs baseline: 13.3951x; 1.0700x over previous
import functools

import jax
import jax.numpy as jnp
from jax import lax
from jax.experimental import pallas as pl
from jax.experimental.pallas import tpu as pltpu
from jax.experimental.pallas import tpu_sc as plsc

N = 100000
NP = 100096
E = 1600000
HID = 32
HALF = 16
CHUNK = 128
NCH = 12544
PAD = NCH * CHUNK - E
NT = 16
CPT = NCH // NT
GRP = 8
NGRP = CPT // GRP
STRIPE = NP // NT
ZROWS = 368
NZC = STRIPE // ZROWS
DUMP = N

_mesh = plsc.VectorSubcoreMesh(core_axis_name="c", subcore_axis_name="s",
                               num_cores=2, num_subcores=NT)


def _counts_body(dst_hbm, ones_hbm, cnt_hbm, acc, dstb, ones_v, ssem):
    cid = lax.axis_index("c")
    tid = lax.axis_index("s")
    pltpu.sync_copy(ones_hbm, ones_v)
    rel = cid * 2

    @pl.loop(0, NZC)
    def _zero(k):
        pltpu.sync_copy(ones_v.at[pl.ds(CHUNK, ZROWS)],
                        acc.at[pl.ds(tid * STRIPE + k * ZROWS, ZROWS)])

    plsc.subcore_barrier()

    @pl.loop(0, NGRP)
    def _grp(g):
        base = tid * CPT + g * GRP
        pltpu.sync_copy(dst_hbm.at[rel, pl.ds(base, GRP)], dstb)
        cps = [pltpu.async_copy(ones_v.at[pl.ds(0, CHUNK)],
                                acc.at[dstb.at[j]], ssem, add=True)
               for j in range(GRP)]
        for cp in cps:
            cp.wait()

    plsc.subcore_barrier()
    pltpu.sync_copy(acc.at[pl.ds(tid * STRIPE, STRIPE)],
                    cnt_hbm.at[cid, pl.ds(tid * STRIPE, STRIPE)])


def _spmm_body(tbl_hbm, src_hbm, dst_hbm, ones_hbm, agg_hbm,
               acc, srcb, dstb, rows, zed, gsem, ssems, isem):
    cid = lax.axis_index("c")
    tid = lax.axis_index("s")
    pltpu.sync_copy(ones_hbm.at[pl.ds(CHUNK, ZROWS)], zed)

    def fire_idx(r, g, p):
        base = tid * CPT + g * GRP
        pltpu.async_copy(src_hbm.at[r, cid, pl.ds(base, GRP)], srcb.at[p],
                         isem)
        pltpu.async_copy(dst_hbm.at[r, pl.ds(base, GRP)], dstb.at[p], isem)

    def wait_idx(r, g, p):
        base = tid * CPT + g * GRP
        pltpu.make_async_copy(src_hbm.at[r, cid, pl.ds(base, GRP)],
                              srcb.at[p], isem).wait()
        pltpu.make_async_copy(dst_hbm.at[r, pl.ds(base, GRP)], dstb.at[p],
                              isem).wait()

    for r in range(3):
        @pl.loop(0, NZC)
        def _zero(k):
            pltpu.sync_copy(zed, acc.at[pl.ds(tid * STRIPE + k * ZROWS, ZROWS)])

        plsc.subcore_barrier()
        fire_idx(r, 0, 0)

        @pl.loop(0, NGRP)
        def _grp(g):
            p = lax.rem(g, 2)
            wait_idx(r, g, p)

            @pl.when(g > 0)
            def _drain_prev():
                for j in range(GRP):
                    pltpu.make_async_copy(rows.at[j], acc.at[dstb.at[p, j]],
                                          ssems.at[j]).wait()

            @pl.when(g < NGRP - 1)
            def _pf():
                fire_idx(r, g + 1, 1 - p)

            cps = [pltpu.async_copy(tbl_hbm.at[srcb.at[p, j]], rows.at[j],
                                    gsem)
                   for j in range(GRP)]
            for cp in cps:
                cp.wait()
            for j in range(GRP):
                pltpu.async_copy(rows.at[j], acc.at[dstb.at[p, j]],
                                 ssems.at[j], add=True)

        pfin = lax.rem(NGRP - 1, 2)
        for j in range(GRP):
            pltpu.make_async_copy(rows.at[j], acc.at[dstb.at[pfin, j]],
                                  ssems.at[j]).wait()

        plsc.subcore_barrier()
        pltpu.sync_copy(acc.at[pl.ds(tid * STRIPE, STRIPE)],
                        agg_hbm.at[pl.ds(tid * STRIPE, STRIPE),
                                   pl.ds(2 * HALF * r + HALF * cid, HALF)])
        plsc.subcore_barrier()


_counts_call = pl.kernel(
    _counts_body,
    out_type=jax.ShapeDtypeStruct((2, NP, HALF), jnp.float32),
    mesh=_mesh,
    compiler_params=pltpu.CompilerParams(use_tc_tiling_on_sc=False),
    scratch_types=[
        pltpu.VMEM_SHARED((NP, HALF), jnp.float32),
        pltpu.VMEM((GRP, CHUNK), jnp.int32),
        pltpu.VMEM((CHUNK + ZROWS, HALF), jnp.float32),
        pltpu.SemaphoreType.DMA,
    ],
    name="sc_counts",
)

_spmm_call = pl.kernel(
    _spmm_body,
    out_type=jax.ShapeDtypeStruct((NP, 6 * HALF), jnp.float32),
    mesh=_mesh,
    compiler_params=pltpu.CompilerParams(use_tc_tiling_on_sc=False),
    scratch_types=[
        pltpu.VMEM_SHARED((NP, HALF), jnp.float32),
        pltpu.VMEM((2, GRP, CHUNK), jnp.int32),
        pltpu.VMEM((2, GRP, CHUNK), jnp.int32),
        pltpu.VMEM((GRP, CHUNK, HALF), jnp.float32),
        pltpu.VMEM((ZROWS, HALF), jnp.float32),
        pltpu.SemaphoreType.DMA,
        pltpu.SemaphoreType.DMA((GRP,)),
        pltpu.SemaphoreType.DMA,
    ],
    name="sc_spmm",
)


def _tc_body(agg_ref, h_ref, cnt_ref, m_ref, w_ref, b_ref, out_ref, *, mode):
    inv0 = 1.0 / jnp.maximum(cnt_ref[0], 1.0)
    inv2 = 1.0 / jnp.maximum(cnt_ref[1], 1.0)
    one = jnp.ones_like(inv0)
    scale = jnp.concatenate([inv0, inv0, one, one, inv2, inv2],
                            axis=-1)
    za = agg_ref[...] * scale
    h = h_ref[...]
    o = (jnp.dot(za, m_ref[...], preferred_element_type=jnp.float32)
         + jnp.dot(h, w_ref[...], preferred_element_type=jnp.float32)
         + b_ref[...])
    if mode == "head":
        hn = o
    else:
        hn = h + jnp.maximum(o, 0.0)
    if mode == "final":
        hn = jnp.maximum(hn, 0.0)
    out_ref[...] = hn


_TCB = 1088


def _tc_call(agg, h, cnt, m, w, bias, mode):
    grid = (NP // _TCB,)
    in_specs = [
        pl.BlockSpec((_TCB, 6 * HALF), lambda i: (i, 0)),
        pl.BlockSpec((_TCB, HID), lambda i: (i, 0)),
        pl.BlockSpec((2, _TCB, HALF), lambda i: (0, i, 0)),
        pl.BlockSpec((6 * HALF, HID), lambda i: (0, 0)),
        pl.BlockSpec((HID, HID), lambda i: (0, 0)),
        pl.BlockSpec((1, HID), lambda i: (0, 0)),
    ]
    return pl.pallas_call(
        functools.partial(_tc_body, mode=mode),
        grid=grid,
        in_specs=in_specs,
        out_specs=pl.BlockSpec((_TCB, HID), lambda i: (i, 0)),
        out_shape=jax.ShapeDtypeStruct((NP, HID), jnp.float32),
        name=f"tc_dense_{mode}",
    )(agg, h, cnt, m, w, bias)


def _prep_edges(ei):
    src = ei[0].astype(jnp.int32)
    dst = ei[1].astype(jnp.int32)
    src = jnp.concatenate([src, jnp.zeros((PAD,), jnp.int32)])
    dst = jnp.concatenate([dst, jnp.full((PAD,), DUMP, jnp.int32)])
    src2 = jnp.stack([2 * src, 2 * src + 1]).reshape(2, NCH, CHUNK)
    return src2, dst.reshape(NCH, CHUNK)


def _pad_rows(w):
    return jnp.pad(w, ((0, HID - w.shape[0]), (0, 0)))


def kernel(x_stroke, edge_index_intersects, edge_index_temp_previous,
           edge_index_represented_by, W_msg_head, W_self_head, b_head,
           W_msg, W_self, b):
    f32 = jnp.float32
    prepped = [_prep_edges(e) for e in (edge_index_intersects,
                                        edge_index_temp_previous,
                                        edge_index_represented_by)]
    src_all = jnp.stack([p[0] for p in prepped])
    dst_all = jnp.stack([p[1] for p in prepped])

    ones_z = jnp.concatenate([jnp.ones((CHUNK, HALF), f32),
                              jnp.zeros((ZROWS, HALF), f32)])

    m_head = jnp.concatenate([_pad_rows(W_msg_head[0].astype(f32)),
                              _pad_rows(W_msg_head[1].astype(f32)),
                              _pad_rows(W_msg_head[2].astype(f32))])
    w_head = _pad_rows(W_self_head.astype(f32).sum(0))
    b_hd = b_head.astype(f32).sum(0).reshape(1, HID)
    ms = [jnp.concatenate([W_msg[l, 0], W_msg[l, 1],
                           W_msg[l, 2]]).astype(f32) for l in range(4)]
    ws = [W_self[l].sum(0).astype(f32) for l in range(4)]
    bs = [b[l].astype(f32).sum(0).reshape(1, HID) for l in range(4)]

    h = jnp.pad(x_stroke.astype(f32),
                ((0, NP - N), (0, HID - x_stroke.shape[1])))

    cnt = _counts_call(dst_all, ones_z)

    agg = _spmm_call(h.reshape(2 * NP, HALF), src_all, dst_all, ones_z)
    h = _tc_call(agg, h, cnt, m_head, w_head, b_hd, "head")
    for l in range(4):
        agg = _spmm_call(h.reshape(2 * NP, HALF), src_all, dst_all, ones_z)
        mode = "final" if l == 3 else "mid"
        h = _tc_call(agg, h, cnt, ms[l], ws[l], bs[l], mode)
    return h[:N]


# speedup vs baseline: 15.4160x; 1.1509x over previous
import functools

import jax
import jax.numpy as jnp
from jax import lax
from jax.experimental import pallas as pl
from jax.experimental.pallas import tpu as pltpu
from jax.experimental.pallas import tpu_sc as plsc

N = 100000
NP = 100096
E = 1600000
HID = 32
HALF = 16
CHUNK = 128
NCH = 12544
PAD = NCH * CHUNK - E
NT = 16
CPT = NCH // NT
GRP = 8
NGRP = CPT // GRP
STRIPE = NP // NT
ZROWS = 368
NZC = STRIPE // ZROWS
DUMP = N

_mesh = plsc.VectorSubcoreMesh(core_axis_name="c", subcore_axis_name="s",
                               num_cores=2, num_subcores=NT)


def _counts_body(dst_hbm, ones_hbm, cnt_hbm, acc, dstb, ones_v, ssem):
    cid = lax.axis_index("c")
    tid = lax.axis_index("s")
    pltpu.sync_copy(ones_hbm, ones_v)
    rel = cid * 2

    @pl.loop(0, NZC)
    def _zero(k):
        pltpu.sync_copy(ones_v.at[pl.ds(CHUNK, ZROWS)],
                        acc.at[pl.ds(tid * STRIPE + k * ZROWS, ZROWS)])

    plsc.subcore_barrier()

    @pl.loop(0, NGRP)
    def _grp(g):
        base = tid * CPT + g * GRP
        pltpu.sync_copy(dst_hbm.at[rel, pl.ds(base, GRP)], dstb)
        cps = [pltpu.async_copy(ones_v.at[pl.ds(0, CHUNK)],
                                acc.at[dstb.at[j]], ssem, add=True)
               for j in range(GRP)]
        for cp in cps:
            cp.wait()

    plsc.subcore_barrier()
    pltpu.sync_copy(acc.at[pl.ds(tid * STRIPE, STRIPE)],
                    cnt_hbm.at[cid, pl.ds(tid * STRIPE, STRIPE)])


def _spmm_body(tbl_hbm, src_hbm, dst_hbm, ones_hbm, agg_hbm,
               acc, srcb, dstb, rows, zed, gsems, ssems, isem):
    cid = lax.axis_index("c")
    tid = lax.axis_index("s")
    pltpu.sync_copy(ones_hbm.at[pl.ds(CHUNK, ZROWS)], zed)

    def fire_idx(r, g, p):
        base = tid * CPT + g * GRP
        pltpu.async_copy(src_hbm.at[r, cid, pl.ds(base, GRP)], srcb.at[p],
                         isem)
        pltpu.async_copy(dst_hbm.at[r, pl.ds(base, GRP)], dstb.at[p], isem)

    def wait_idx(r, g, p):
        base = tid * CPT + g * GRP
        pltpu.make_async_copy(src_hbm.at[r, cid, pl.ds(base, GRP)],
                              srcb.at[p], isem).wait()
        pltpu.make_async_copy(dst_hbm.at[r, pl.ds(base, GRP)], dstb.at[p],
                              isem).wait()

    for r in range(3):
        @pl.loop(0, NZC)
        def _zero(k):
            pltpu.sync_copy(zed, acc.at[pl.ds(tid * STRIPE + k * ZROWS, ZROWS)])

        plsc.subcore_barrier()
        fire_idx(r, 0, 0)

        @pl.loop(0, NGRP)
        def _grp(g):
            p = lax.rem(g, 2)
            wait_idx(r, g, p)

            @pl.when(g > 0)
            def _drain_prev():
                for j in range(GRP):
                    pltpu.make_async_copy(rows.at[j], acc.at[dstb.at[p, j]],
                                          ssems.at[j]).wait()

            @pl.when(g < NGRP - 1)
            def _pf():
                fire_idx(r, g + 1, 1 - p)

            cps = [pltpu.async_copy(tbl_hbm.at[srcb.at[p, j]], rows.at[j],
                                    gsems.at[j])
                   for j in range(GRP)]
            for j in range(GRP):
                cps[j].wait()
                pltpu.async_copy(rows.at[j], acc.at[dstb.at[p, j]],
                                 ssems.at[j], add=True)

        pfin = lax.rem(NGRP - 1, 2)
        for j in range(GRP):
            pltpu.make_async_copy(rows.at[j], acc.at[dstb.at[pfin, j]],
                                  ssems.at[j]).wait()

        plsc.subcore_barrier()
        pltpu.sync_copy(acc.at[pl.ds(tid * STRIPE, STRIPE)],
                        agg_hbm.at[pl.ds(tid * STRIPE, STRIPE),
                                   pl.ds(2 * HALF * r + HALF * cid, HALF)])
        plsc.subcore_barrier()


_counts_call = pl.kernel(
    _counts_body,
    out_type=jax.ShapeDtypeStruct((2, NP, HALF), jnp.float32),
    mesh=_mesh,
    compiler_params=pltpu.CompilerParams(use_tc_tiling_on_sc=False),
    scratch_types=[
        pltpu.VMEM_SHARED((NP, HALF), jnp.float32),
        pltpu.VMEM((GRP, CHUNK), jnp.int32),
        pltpu.VMEM((CHUNK + ZROWS, HALF), jnp.float32),
        pltpu.SemaphoreType.DMA,
    ],
    name="sc_counts",
)

_spmm_call = pl.kernel(
    _spmm_body,
    out_type=jax.ShapeDtypeStruct((NP, 6 * HALF), jnp.float32),
    mesh=_mesh,
    compiler_params=pltpu.CompilerParams(use_tc_tiling_on_sc=False),
    scratch_types=[
        pltpu.VMEM_SHARED((NP, HALF), jnp.float32),
        pltpu.VMEM((2, GRP, CHUNK), jnp.int32),
        pltpu.VMEM((2, GRP, CHUNK), jnp.int32),
        pltpu.VMEM((GRP, CHUNK, HALF), jnp.float32),
        pltpu.VMEM((ZROWS, HALF), jnp.float32),
        pltpu.SemaphoreType.DMA((GRP,)),
        pltpu.SemaphoreType.DMA((GRP,)),
        pltpu.SemaphoreType.DMA,
    ],
    name="sc_spmm",
)


def _tc_body(agg_ref, h_ref, cnt_ref, m_ref, w_ref, b_ref, out_ref, *, mode):
    inv0 = 1.0 / jnp.maximum(cnt_ref[0], 1.0)
    inv2 = 1.0 / jnp.maximum(cnt_ref[1], 1.0)
    one = jnp.ones_like(inv0)
    scale = jnp.concatenate([inv0, inv0, one, one, inv2, inv2],
                            axis=-1)
    za = agg_ref[...] * scale
    h = h_ref[...]
    o = (jnp.dot(za, m_ref[...], preferred_element_type=jnp.float32)
         + jnp.dot(h, w_ref[...], preferred_element_type=jnp.float32)
         + b_ref[...])
    if mode == "head":
        hn = o
    else:
        hn = h + jnp.maximum(o, 0.0)
    if mode == "final":
        hn = jnp.maximum(hn, 0.0)
    out_ref[...] = hn


_TCB = 1088


def _tc_call(agg, h, cnt, m, w, bias, mode):
    grid = (NP // _TCB,)
    in_specs = [
        pl.BlockSpec((_TCB, 6 * HALF), lambda i: (i, 0)),
        pl.BlockSpec((_TCB, HID), lambda i: (i, 0)),
        pl.BlockSpec((2, _TCB, HALF), lambda i: (0, i, 0)),
        pl.BlockSpec((6 * HALF, HID), lambda i: (0, 0)),
        pl.BlockSpec((HID, HID), lambda i: (0, 0)),
        pl.BlockSpec((1, HID), lambda i: (0, 0)),
    ]
    return pl.pallas_call(
        functools.partial(_tc_body, mode=mode),
        grid=grid,
        in_specs=in_specs,
        out_specs=pl.BlockSpec((_TCB, HID), lambda i: (i, 0)),
        out_shape=jax.ShapeDtypeStruct((NP, HID), jnp.float32),
        name=f"tc_dense_{mode}",
    )(agg, h, cnt, m, w, bias)


def _prep_edges(ei):
    src = ei[0].astype(jnp.int32)
    dst = ei[1].astype(jnp.int32)
    src = jnp.concatenate([src, jnp.zeros((PAD,), jnp.int32)])
    dst = jnp.concatenate([dst, jnp.full((PAD,), DUMP, jnp.int32)])
    src2 = jnp.stack([2 * src, 2 * src + 1]).reshape(2, NCH, CHUNK)
    return src2, dst.reshape(NCH, CHUNK)


def _pad_rows(w):
    return jnp.pad(w, ((0, HID - w.shape[0]), (0, 0)))


def kernel(x_stroke, edge_index_intersects, edge_index_temp_previous,
           edge_index_represented_by, W_msg_head, W_self_head, b_head,
           W_msg, W_self, b):
    f32 = jnp.float32
    prepped = [_prep_edges(e) for e in (edge_index_intersects,
                                        edge_index_temp_previous,
                                        edge_index_represented_by)]
    src_all = jnp.stack([p[0] for p in prepped])
    dst_all = jnp.stack([p[1] for p in prepped])

    ones_z = jnp.concatenate([jnp.ones((CHUNK, HALF), f32),
                              jnp.zeros((ZROWS, HALF), f32)])

    m_head = jnp.concatenate([_pad_rows(W_msg_head[0].astype(f32)),
                              _pad_rows(W_msg_head[1].astype(f32)),
                              _pad_rows(W_msg_head[2].astype(f32))])
    w_head = _pad_rows(W_self_head.astype(f32).sum(0))
    b_hd = b_head.astype(f32).sum(0).reshape(1, HID)
    ms = [jnp.concatenate([W_msg[l, 0], W_msg[l, 1],
                           W_msg[l, 2]]).astype(f32) for l in range(4)]
    ws = [W_self[l].sum(0).astype(f32) for l in range(4)]
    bs = [b[l].astype(f32).sum(0).reshape(1, HID) for l in range(4)]

    h = jnp.pad(x_stroke.astype(f32),
                ((0, NP - N), (0, HID - x_stroke.shape[1])))

    cnt = _counts_call(dst_all, ones_z)

    agg = _spmm_call(h.reshape(2 * NP, HALF), src_all, dst_all, ones_z)
    h = _tc_call(agg, h, cnt, m_head, w_head, b_hd, "head")
    for l in range(4):
        agg = _spmm_call(h.reshape(2 * NP, HALF), src_all, dst_all, ones_z)
        mode = "final" if l == 3 else "mid"
        h = _tc_call(agg, h, cnt, ms[l], ws[l], bs[l], mode)
    return h[:N]


# speedup vs baseline: 16.3663x; 1.0616x over previous
import functools

import jax
import jax.numpy as jnp
from jax import lax
from jax.experimental import pallas as pl
from jax.experimental.pallas import tpu as pltpu
from jax.experimental.pallas import tpu_sc as plsc

N = 100000
NP = 100096
E = 1600000
HID = 32
HALF = 16
CHUNK = 128
NCH = 12544
PAD = NCH * CHUNK - E
NT = 16
CPT = NCH // NT
GRP = 8
NGRP = CPT // GRP
STRIPE = NP // NT
ZROWS = 368
NZC = STRIPE // ZROWS
DUMP = N
NCH2 = NCH // 2
CPT2 = NCH2 // NT
NGRP2 = CPT2 // GRP

_mesh = plsc.VectorSubcoreMesh(core_axis_name="c", subcore_axis_name="s",
                               num_cores=2, num_subcores=NT)


def _relation_pass(tbl_hbm, src_hbm, dst_hbm, agg_hbm, acc, srcb, dstb,
                   rows, zed, gsems, ssems, isem, tid, cid,
                   r, slab, base0, ngrp, col):
    cpt = ngrp * GRP

    def fire_idx(g, p):
        base = base0 + tid * cpt + g * GRP
        pltpu.async_copy(src_hbm.at[r, slab, pl.ds(base, GRP)], srcb.at[p],
                         isem)
        pltpu.async_copy(dst_hbm.at[r, pl.ds(base, GRP)], dstb.at[p], isem)

    def wait_idx(g, p):
        base = base0 + tid * cpt + g * GRP
        pltpu.make_async_copy(src_hbm.at[r, slab, pl.ds(base, GRP)],
                              srcb.at[p], isem).wait()
        pltpu.make_async_copy(dst_hbm.at[r, pl.ds(base, GRP)], dstb.at[p],
                              isem).wait()

    @pl.loop(0, NZC)
    def _zero(k):
        pltpu.sync_copy(zed, acc.at[pl.ds(tid * STRIPE + k * ZROWS, ZROWS)])

    plsc.subcore_barrier()
    fire_idx(0, 0)

    @pl.loop(0, ngrp)
    def _grp(g):
        p = lax.rem(g, 2)
        wait_idx(g, p)

        @pl.when(g > 0)
        def _drain_prev():
            for j in range(GRP):
                pltpu.make_async_copy(rows.at[j], acc.at[dstb.at[p, j]],
                                      ssems.at[j]).wait()

        @pl.when(g < ngrp - 1)
        def _pf():
            fire_idx(g + 1, 1 - p)

        cps = [pltpu.async_copy(tbl_hbm.at[srcb.at[p, j]], rows.at[j],
                                gsems.at[j])
               for j in range(GRP)]
        for j in range(GRP):
            cps[j].wait()
            pltpu.async_copy(rows.at[j], acc.at[dstb.at[p, j]],
                             ssems.at[j], add=True)

    pfin = lax.rem(ngrp - 1, 2)
    for j in range(GRP):
        pltpu.make_async_copy(rows.at[j], acc.at[dstb.at[pfin, j]],
                              ssems.at[j]).wait()

    plsc.subcore_barrier()
    pltpu.sync_copy(acc.at[pl.ds(tid * STRIPE, STRIPE)],
                    agg_hbm.at[pl.ds(tid * STRIPE, STRIPE),
                               pl.ds(col, HALF)])
    plsc.subcore_barrier()


def _spmm_body(tbl_hbm, src_hbm, dst_hbm, ones_hbm, agg_hbm,
               acc, srcb, dstb, rows, zed, gsems, ssems, isem):
    cid = lax.axis_index("c")
    tid = lax.axis_index("s")
    pltpu.sync_copy(ones_hbm.at[pl.ds(CHUNK, ZROWS)], zed)
    for r in range(3):
        _relation_pass(tbl_hbm, src_hbm, dst_hbm, agg_hbm, acc, srcb, dstb,
                       rows, zed, gsems, ssems, isem, tid, cid,
                       r, cid, 0, NGRP, 2 * HALF * r + HALF * cid)


def _head_body(tbl_hbm, src_hbm, dst_hbm, ones_hbm, agg_hbm, cnt_hbm,
               acc, srcb, dstb, rows, zed, gsems, ssems, isem):
    cid = lax.axis_index("c")
    tid = lax.axis_index("s")
    pltpu.sync_copy(ones_hbm.at[pl.ds(CHUNK, ZROWS)], zed)
    for r in range(3):
        _relation_pass(tbl_hbm, src_hbm, dst_hbm, agg_hbm, acc, srcb, dstb,
                       rows, zed, gsems, ssems, isem, tid, cid,
                       r, 0, cid * NCH2, NGRP2, 2 * HALF * r + HALF * cid)

    rel = cid * 2

    @pl.loop(0, NZC)
    def _zero(k):
        pltpu.sync_copy(zed, acc.at[pl.ds(tid * STRIPE + k * ZROWS, ZROWS)])

    pltpu.sync_copy(ones_hbm.at[pl.ds(0, CHUNK)], rows.at[0])
    plsc.subcore_barrier()

    @pl.loop(0, NGRP)
    def _grp(g):
        base = tid * CPT + g * GRP
        pltpu.sync_copy(dst_hbm.at[rel, pl.ds(base, GRP)], dstb.at[0])
        cps = [pltpu.async_copy(rows.at[0], acc.at[dstb.at[0, j]],
                                ssems.at[j], add=True)
               for j in range(GRP)]
        for cp in cps:
            cp.wait()

    plsc.subcore_barrier()
    pltpu.sync_copy(acc.at[pl.ds(tid * STRIPE, STRIPE)],
                    cnt_hbm.at[cid, pl.ds(tid * STRIPE, STRIPE)])


_head_call = pl.kernel(
    _head_body,
    out_type=[jax.ShapeDtypeStruct((NP, 6 * HALF), jnp.float32),
              jax.ShapeDtypeStruct((2, NP, HALF), jnp.float32)],
    mesh=_mesh,
    compiler_params=pltpu.CompilerParams(use_tc_tiling_on_sc=False),
    scratch_types=[
        pltpu.VMEM_SHARED((NP, HALF), jnp.float32),
        pltpu.VMEM((2, GRP, CHUNK), jnp.int32),
        pltpu.VMEM((2, GRP, CHUNK), jnp.int32),
        pltpu.VMEM((GRP, CHUNK, HALF), jnp.float32),
        pltpu.VMEM((ZROWS, HALF), jnp.float32),
        pltpu.SemaphoreType.DMA((GRP,)),
        pltpu.SemaphoreType.DMA((GRP,)),
        pltpu.SemaphoreType.DMA,
    ],
    name="sc_head",
)

_spmm_call = pl.kernel(
    _spmm_body,
    out_type=jax.ShapeDtypeStruct((NP, 6 * HALF), jnp.float32),
    mesh=_mesh,
    compiler_params=pltpu.CompilerParams(use_tc_tiling_on_sc=False),
    scratch_types=[
        pltpu.VMEM_SHARED((NP, HALF), jnp.float32),
        pltpu.VMEM((2, GRP, CHUNK), jnp.int32),
        pltpu.VMEM((2, GRP, CHUNK), jnp.int32),
        pltpu.VMEM((GRP, CHUNK, HALF), jnp.float32),
        pltpu.VMEM((ZROWS, HALF), jnp.float32),
        pltpu.SemaphoreType.DMA((GRP,)),
        pltpu.SemaphoreType.DMA((GRP,)),
        pltpu.SemaphoreType.DMA,
    ],
    name="sc_spmm",
)


def _tc_body(agg_ref, h_ref, cnt_ref, m_ref, w_ref, b_ref, out_ref, *, mode):
    inv0 = 1.0 / jnp.maximum(cnt_ref[0], 1.0)
    inv2 = 1.0 / jnp.maximum(cnt_ref[1], 1.0)
    a = agg_ref[...]
    if mode == "head":
        a0 = (a[:, 0:HALF] + a[:, HALF:2 * HALF]) * inv0
        a1 = a[:, 2 * HALF:3 * HALF] + a[:, 3 * HALF:4 * HALF]
        a2 = (a[:, 4 * HALF:5 * HALF] + a[:, 5 * HALF:6 * HALF]) * inv2
        za = jnp.concatenate([a0, a1, a2], axis=-1)
    else:
        one = jnp.ones_like(inv0)
        scale = jnp.concatenate([inv0, inv0, one, one, inv2, inv2],
                                axis=-1)
        za = a * scale
    h = h_ref[...]
    o = (jnp.dot(za, m_ref[...], preferred_element_type=jnp.float32)
         + jnp.dot(h, w_ref[...], preferred_element_type=jnp.float32)
         + b_ref[...])
    if mode == "head":
        hn = o
    else:
        hn = h + jnp.maximum(o, 0.0)
    if mode == "final":
        hn = jnp.maximum(hn, 0.0)
    out_ref[...] = hn


_TCB = 1088


def _tc_call(agg, h, cnt, m, w, bias, mode):
    grid = (NP // _TCB,)
    in_specs = [
        pl.BlockSpec((_TCB, 6 * HALF), lambda i: (i, 0)),
        pl.BlockSpec((_TCB, HID), lambda i: (i, 0)),
        pl.BlockSpec((2, _TCB, HALF), lambda i: (0, i, 0)),
        pl.BlockSpec((m.shape[0], HID), lambda i: (0, 0)),
        pl.BlockSpec((HID, HID), lambda i: (0, 0)),
        pl.BlockSpec((1, HID), lambda i: (0, 0)),
    ]
    return pl.pallas_call(
        functools.partial(_tc_body, mode=mode),
        grid=grid,
        in_specs=in_specs,
        out_specs=pl.BlockSpec((_TCB, HID), lambda i: (i, 0)),
        out_shape=jax.ShapeDtypeStruct((NP, HID), jnp.float32),
        name=f"tc_dense_{mode}",
    )(agg, h, cnt, m, w, bias)


def _prep_edges(ei):
    src = ei[0].astype(jnp.int32)
    dst = ei[1].astype(jnp.int32)
    src = jnp.concatenate([src, jnp.zeros((PAD,), jnp.int32)])
    dst = jnp.concatenate([dst, jnp.full((PAD,), DUMP, jnp.int32)])
    src2 = jnp.stack([2 * src, 2 * src + 1]).reshape(2, NCH, CHUNK)
    return src2, dst.reshape(NCH, CHUNK)


def _pad_rows(w):
    return jnp.pad(w, ((0, HID - w.shape[0]), (0, 0)))


def _pad16(w):
    return jnp.pad(w, ((0, HALF - w.shape[0]), (0, 0)))


def kernel(x_stroke, edge_index_intersects, edge_index_temp_previous,
           edge_index_represented_by, W_msg_head, W_self_head, b_head,
           W_msg, W_self, b):
    f32 = jnp.float32
    prepped = [_prep_edges(e) for e in (edge_index_intersects,
                                        edge_index_temp_previous,
                                        edge_index_represented_by)]
    src_all = jnp.stack([p[0] for p in prepped])
    dst_all = jnp.stack([p[1] for p in prepped])

    ones_z = jnp.concatenate([jnp.ones((CHUNK, HALF), f32),
                              jnp.zeros((ZROWS, HALF), f32)])

    m_head = jnp.concatenate([_pad16(W_msg_head[0].astype(f32)),
                              _pad16(W_msg_head[1].astype(f32)),
                              _pad16(W_msg_head[2].astype(f32))])
    w_head = _pad_rows(W_self_head.astype(f32).sum(0))
    b_hd = b_head.astype(f32).sum(0).reshape(1, HID)
    ms = [jnp.concatenate([W_msg[l, 0], W_msg[l, 1],
                           W_msg[l, 2]]).astype(f32) for l in range(4)]
    ws = [W_self[l].sum(0).astype(f32) for l in range(4)]
    bs = [b[l].astype(f32).sum(0).reshape(1, HID) for l in range(4)]

    h = jnp.pad(x_stroke.astype(f32),
                ((0, NP - N), (0, HID - x_stroke.shape[1])))

    agg, cnt = _head_call(h.reshape(2 * NP, HALF), src_all, dst_all, ones_z)
    h = _tc_call(agg, h, cnt, m_head, w_head, b_hd, "head")
    for l in range(4):
        agg = _spmm_call(h.reshape(2 * NP, HALF), src_all, dst_all, ones_z)
        mode = "final" if l == 3 else "mid"
        h = _tc_call(agg, h, cnt, ms[l], ws[l], bs[l], mode)
    return h[:N]


# speedup vs baseline: 17.2893x; 1.0564x over previous
import functools

import jax
import jax.numpy as jnp
from jax import lax
from jax.experimental import pallas as pl
from jax.experimental.pallas import tpu as pltpu
from jax.experimental.pallas import tpu_sc as plsc

N = 100000
NP = 100096
E = 1600000
HID = 32
HALF = 16
CHUNK = 128
NCH = 12544
PAD = NCH * CHUNK - E
NT = 16
CPT = NCH // NT
GRP = 8
NGRP = CPT // GRP
STRIPE = NP // NT
ZROWS = 368
NZC = STRIPE // ZROWS
DUMP = N
NCH2 = NCH // 2
CPT2 = NCH2 // NT
NGRP2 = CPT2 // GRP

_mesh = plsc.VectorSubcoreMesh(core_axis_name="c", subcore_axis_name="s",
                               num_cores=2, num_subcores=NT)


def _relation_pass(tbl_hbm, src_hbm, dst_hbm, agg_hbm, acc, srcb, dstb,
                   rows, zed, gsems, ssems, isem, tid, cid,
                   r, slab, base0, ngrp, col):
    cpt = ngrp * GRP

    def fire_idx(g, p):
        base = base0 + tid * cpt + g * GRP
        pltpu.async_copy(src_hbm.at[r, slab, pl.ds(base, GRP)], srcb.at[p],
                         isem)
        pltpu.async_copy(dst_hbm.at[r, pl.ds(base, GRP)], dstb.at[p], isem)

    def wait_idx(g, p):
        base = base0 + tid * cpt + g * GRP
        pltpu.make_async_copy(src_hbm.at[r, slab, pl.ds(base, GRP)],
                              srcb.at[p], isem).wait()
        pltpu.make_async_copy(dst_hbm.at[r, pl.ds(base, GRP)], dstb.at[p],
                              isem).wait()

    @pl.loop(0, NZC)
    def _zero(k):
        pltpu.sync_copy(zed, acc.at[pl.ds(tid * STRIPE + k * ZROWS, ZROWS)])

    plsc.subcore_barrier()
    fire_idx(0, 0)

    @pl.loop(0, ngrp)
    def _grp(g):
        p = lax.rem(g, 2)
        wait_idx(g, p)

        @pl.when(g > 0)
        def _drain_prev():
            for j in range(GRP):
                pltpu.make_async_copy(rows.at[j], acc.at[dstb.at[p, j]],
                                      ssems.at[j]).wait()

        @pl.when(g < ngrp - 1)
        def _pf():
            fire_idx(g + 1, 1 - p)

        cps = [pltpu.async_copy(tbl_hbm.at[srcb.at[p, j]], rows.at[j],
                                gsems.at[j])
               for j in range(GRP)]
        for j in range(GRP):
            cps[j].wait()
            pltpu.async_copy(rows.at[j], acc.at[dstb.at[p, j]],
                             ssems.at[j], add=True)

    pfin = lax.rem(ngrp - 1, 2)
    for j in range(GRP):
        pltpu.make_async_copy(rows.at[j], acc.at[dstb.at[pfin, j]],
                              ssems.at[j]).wait()

    plsc.subcore_barrier()
    pltpu.sync_copy(acc.at[pl.ds(tid * STRIPE, STRIPE)],
                    agg_hbm.at[pl.ds(tid * STRIPE, STRIPE),
                               pl.ds(col, HALF)])
    plsc.subcore_barrier()


def _spmm_body(tbl_hbm, src_hbm, dst_hbm, ones_hbm, agg_hbm,
               acc, srcb, dstb, rows, zed, gsems, ssems, isem):
    cid = lax.axis_index("c")
    tid = lax.axis_index("s")
    pltpu.sync_copy(ones_hbm.at[pl.ds(CHUNK, ZROWS)], zed)
    for r in range(3):
        _relation_pass(tbl_hbm, src_hbm, dst_hbm, agg_hbm, acc, srcb, dstb,
                       rows, zed, gsems, ssems, isem, tid, cid,
                       r, cid, 0, NGRP, 2 * HALF * r + HALF * cid)


def _head_body(tbl_hbm, src_hbm, dst_hbm, ones_hbm, agg_hbm, cnt_hbm,
               acc, srcb, dstb, rows, zed, gsems, ssems, isem):
    cid = lax.axis_index("c")
    tid = lax.axis_index("s")
    pltpu.sync_copy(ones_hbm.at[pl.ds(CHUNK, ZROWS)], zed)
    for r in range(3):
        _relation_pass(tbl_hbm, src_hbm, dst_hbm, agg_hbm, acc, srcb, dstb,
                       rows, zed, gsems, ssems, isem, tid, cid,
                       r, 0, cid * NCH2, NGRP2, 2 * HALF * r + HALF * cid)

    rel = cid * 2

    @pl.loop(0, NZC)
    def _zero(k):
        pltpu.sync_copy(zed, acc.at[pl.ds(tid * STRIPE + k * ZROWS, ZROWS)])

    pltpu.sync_copy(ones_hbm.at[pl.ds(0, CHUNK)], rows.at[0])
    plsc.subcore_barrier()

    @pl.loop(0, NGRP)
    def _grp(g):
        base = tid * CPT + g * GRP
        pltpu.sync_copy(dst_hbm.at[rel, pl.ds(base, GRP)], dstb.at[0])
        cps = [pltpu.async_copy(rows.at[0], acc.at[dstb.at[0, j]],
                                ssems.at[j], add=True)
               for j in range(GRP)]
        for cp in cps:
            cp.wait()

    plsc.subcore_barrier()
    pltpu.sync_copy(acc.at[pl.ds(tid * STRIPE, STRIPE)],
                    cnt_hbm.at[cid, pl.ds(tid * STRIPE, STRIPE)])


_head_call = pl.kernel(
    _head_body,
    out_type=[jax.ShapeDtypeStruct((NP, 8 * HALF), jnp.float32),
              jax.ShapeDtypeStruct((2, NP, HALF), jnp.float32)],
    mesh=_mesh,
    compiler_params=pltpu.CompilerParams(use_tc_tiling_on_sc=False),
    scratch_types=[
        pltpu.VMEM_SHARED((NP, HALF), jnp.float32),
        pltpu.VMEM((2, GRP, CHUNK), jnp.int32),
        pltpu.VMEM((2, GRP, CHUNK), jnp.int32),
        pltpu.VMEM((GRP, CHUNK, HALF), jnp.float32),
        pltpu.VMEM((ZROWS, HALF), jnp.float32),
        pltpu.SemaphoreType.DMA((GRP,)),
        pltpu.SemaphoreType.DMA((GRP,)),
        pltpu.SemaphoreType.DMA,
    ],
    name="sc_head",
)

_spmm_call = pl.kernel(
    _spmm_body,
    out_type=jax.ShapeDtypeStruct((NP, 8 * HALF), jnp.float32),
    mesh=_mesh,
    compiler_params=pltpu.CompilerParams(use_tc_tiling_on_sc=False),
    scratch_types=[
        pltpu.VMEM_SHARED((NP, HALF), jnp.float32),
        pltpu.VMEM((2, GRP, CHUNK), jnp.int32),
        pltpu.VMEM((2, GRP, CHUNK), jnp.int32),
        pltpu.VMEM((GRP, CHUNK, HALF), jnp.float32),
        pltpu.VMEM((ZROWS, HALF), jnp.float32),
        pltpu.SemaphoreType.DMA((GRP,)),
        pltpu.SemaphoreType.DMA((GRP,)),
        pltpu.SemaphoreType.DMA,
    ],
    name="sc_spmm",
)


def _tc_head_body(agg_ref, h_ref, cnt_ref, m_ref, w_ref, b_ref,
                  out_ref, scale_ref):
    inv0 = 1.0 / jnp.maximum(cnt_ref[0], 1.0)
    inv2 = 1.0 / jnp.maximum(cnt_ref[1], 1.0)
    one = jnp.ones_like(inv0)
    scale_ref[...] = jnp.concatenate([inv0, inv0, one, one, inv2, inv2],
                                     axis=-1)
    a = agg_ref[...]
    a0 = (a[:, 0:HALF] + a[:, HALF:2 * HALF]) * inv0
    a1 = a[:, 2 * HALF:3 * HALF] + a[:, 3 * HALF:4 * HALF]
    a2 = (a[:, 4 * HALF:5 * HALF] + a[:, 5 * HALF:6 * HALF]) * inv2
    za = jnp.concatenate([a0, a1, a2], axis=-1)
    h = h_ref[...]
    out_ref[...] = (jnp.dot(za, m_ref[...], preferred_element_type=jnp.float32)
                    + jnp.dot(h, w_ref[...],
                              preferred_element_type=jnp.float32)
                    + b_ref[...])


def _tc_body(agg_ref, h_ref, scale_ref, m_ref, w_ref, b_ref, out_ref, *,
             mode):
    za = agg_ref[:, :6 * HALF] * scale_ref[...]
    h = h_ref[...]
    o = (jnp.dot(za, m_ref[...], preferred_element_type=jnp.float32)
         + jnp.dot(h, w_ref[...], preferred_element_type=jnp.float32)
         + b_ref[...])
    hn = h + jnp.maximum(o, 0.0)
    if mode == "final":
        hn = jnp.maximum(hn, 0.0)
    out_ref[...] = hn


_TCB = 1088


def _tc_head_call(agg, h, cnt, m, w, bias):
    grid = (NP // _TCB,)
    in_specs = [
        pl.BlockSpec((_TCB, 8 * HALF), lambda i: (i, 0)),
        pl.BlockSpec((_TCB, HID), lambda i: (i, 0)),
        pl.BlockSpec((2, _TCB, HALF), lambda i: (0, i, 0)),
        pl.BlockSpec((3 * HALF, HID), lambda i: (0, 0)),
        pl.BlockSpec((HID, HID), lambda i: (0, 0)),
        pl.BlockSpec((1, HID), lambda i: (0, 0)),
    ]
    return pl.pallas_call(
        _tc_head_body,
        grid=grid,
        in_specs=in_specs,
        out_specs=[pl.BlockSpec((_TCB, HID), lambda i: (i, 0)),
                   pl.BlockSpec((_TCB, 6 * HALF), lambda i: (i, 0))],
        out_shape=[jax.ShapeDtypeStruct((NP, HID), jnp.float32),
                   jax.ShapeDtypeStruct((NP, 6 * HALF), jnp.float32)],
        name="tc_dense_head",
    )(agg, h, cnt, m, w, bias)


def _tc_call(agg, h, scale, m, w, bias, mode):
    grid = (NP // _TCB,)
    in_specs = [
        pl.BlockSpec((_TCB, 8 * HALF), lambda i: (i, 0)),
        pl.BlockSpec((_TCB, HID), lambda i: (i, 0)),
        pl.BlockSpec((_TCB, 6 * HALF), lambda i: (i, 0)),
        pl.BlockSpec((6 * HALF, HID), lambda i: (0, 0)),
        pl.BlockSpec((HID, HID), lambda i: (0, 0)),
        pl.BlockSpec((1, HID), lambda i: (0, 0)),
    ]
    return pl.pallas_call(
        functools.partial(_tc_body, mode=mode),
        grid=grid,
        in_specs=in_specs,
        out_specs=pl.BlockSpec((_TCB, HID), lambda i: (i, 0)),
        out_shape=jax.ShapeDtypeStruct((NP, HID), jnp.float32),
        name=f"tc_dense_{mode}",
    )(agg, h, scale, m, w, bias)


def _prep_edges(ei):
    src = ei[0].astype(jnp.int32)
    dst = ei[1].astype(jnp.int32)
    src = jnp.concatenate([src, jnp.zeros((PAD,), jnp.int32)])
    dst = jnp.concatenate([dst, jnp.full((PAD,), DUMP, jnp.int32)])
    src2 = jnp.stack([2 * src, 2 * src + 1]).reshape(2, NCH, CHUNK)
    return src2, dst.reshape(NCH, CHUNK)


def _pad_rows(w):
    return jnp.pad(w, ((0, HID - w.shape[0]), (0, 0)))


def _pad16(w):
    return jnp.pad(w, ((0, HALF - w.shape[0]), (0, 0)))


def kernel(x_stroke, edge_index_intersects, edge_index_temp_previous,
           edge_index_represented_by, W_msg_head, W_self_head, b_head,
           W_msg, W_self, b):
    f32 = jnp.float32
    prepped = [_prep_edges(e) for e in (edge_index_intersects,
                                        edge_index_temp_previous,
                                        edge_index_represented_by)]
    src_all = jnp.stack([p[0] for p in prepped])
    dst_all = jnp.stack([p[1] for p in prepped])

    ones_z = jnp.concatenate([jnp.ones((CHUNK, HALF), f32),
                              jnp.zeros((ZROWS, HALF), f32)])

    m_head = jnp.concatenate([_pad16(W_msg_head[0].astype(f32)),
                              _pad16(W_msg_head[1].astype(f32)),
                              _pad16(W_msg_head[2].astype(f32))])
    w_head = _pad_rows(W_self_head.astype(f32).sum(0))
    b_hd = b_head.astype(f32).sum(0).reshape(1, HID)
    ms = [jnp.concatenate([W_msg[l, 0], W_msg[l, 1],
                           W_msg[l, 2]]).astype(f32) for l in range(4)]
    ws = [W_self[l].sum(0).astype(f32) for l in range(4)]
    bs = [b[l].astype(f32).sum(0).reshape(1, HID) for l in range(4)]

    h = jnp.pad(x_stroke.astype(f32),
                ((0, NP - N), (0, HID - x_stroke.shape[1])))

    agg, cnt = _head_call(h.reshape(2 * NP, HALF), src_all, dst_all, ones_z)
    h, scale = _tc_head_call(agg, h, cnt, m_head, w_head, b_hd)
    for l in range(4):
        agg = _spmm_call(h.reshape(2 * NP, HALF), src_all, dst_all, ones_z)
        mode = "final" if l == 3 else "mid"
        h = _tc_call(agg, h, scale, ms[l], ws[l], bs[l], mode)
    return h[:N]


# speedup vs baseline: 17.8678x; 1.0335x over previous
import functools

import jax
import jax.numpy as jnp
from jax import lax
from jax.experimental import pallas as pl
from jax.experimental.pallas import tpu as pltpu
from jax.experimental.pallas import tpu_sc as plsc

N = 100000
NP = 100096
E = 1600000
HID = 32
HALF = 16
CHUNK = 128
NCH = 12544
PAD = NCH * CHUNK - E
NT = 16
CPT = NCH // NT
GRP = 8
NGRP = CPT // GRP
STRIPE = NP // NT
ZROWS = 368
NZC = STRIPE // ZROWS
DUMP = N
NCH2 = NCH // 2
CPT2 = NCH2 // NT
NGRP2 = CPT2 // GRP

_mesh = plsc.VectorSubcoreMesh(core_axis_name="c", subcore_axis_name="s",
                               num_cores=2, num_subcores=NT)


def _relation_pass(tbl_hbm, src_hbm, dst_hbm, agg_hbm, acc, srcb, dstb,
                   rows, zed, gsems, ssems, isem, tid, cid,
                   r, slab, base0, ngrp, col):
    cpt = ngrp * GRP

    def fire_idx(g, p):
        base = base0 + tid * cpt + g * GRP
        pltpu.async_copy(src_hbm.at[r, slab, pl.ds(base, GRP)], srcb.at[p],
                         isem)
        pltpu.async_copy(dst_hbm.at[r, pl.ds(base, GRP)], dstb.at[p], isem)

    def wait_idx(g, p):
        base = base0 + tid * cpt + g * GRP
        pltpu.make_async_copy(src_hbm.at[r, slab, pl.ds(base, GRP)],
                              srcb.at[p], isem).wait()
        pltpu.make_async_copy(dst_hbm.at[r, pl.ds(base, GRP)], dstb.at[p],
                              isem).wait()

    zcps = [pltpu.async_copy(zed,
                             acc.at[pl.ds(tid * STRIPE + k * ZROWS, ZROWS)],
                             isem)
            for k in range(NZC)]
    for cp in zcps:
        cp.wait()

    plsc.subcore_barrier()
    fire_idx(0, 0)

    @pl.loop(0, ngrp)
    def _grp(g):
        p = lax.rem(g, 2)
        wait_idx(g, p)

        @pl.when(g > 0)
        def _drain_prev():
            for j in range(GRP):
                pltpu.make_async_copy(rows.at[j], acc.at[dstb.at[p, j]],
                                      ssems.at[j]).wait()

        @pl.when(g < ngrp - 1)
        def _pf():
            fire_idx(g + 1, 1 - p)

        cps = [pltpu.async_copy(tbl_hbm.at[srcb.at[p, j]], rows.at[j],
                                gsems.at[j])
               for j in range(GRP)]
        for j in range(GRP):
            cps[j].wait()
            pltpu.async_copy(rows.at[j], acc.at[dstb.at[p, j]],
                             ssems.at[j], add=True)

    pfin = lax.rem(ngrp - 1, 2)
    for j in range(GRP):
        pltpu.make_async_copy(rows.at[j], acc.at[dstb.at[pfin, j]],
                              ssems.at[j]).wait()

    plsc.subcore_barrier()
    pltpu.sync_copy(acc.at[pl.ds(tid * STRIPE, STRIPE)],
                    agg_hbm.at[pl.ds(tid * STRIPE, STRIPE),
                               pl.ds(col, HALF)])
    plsc.subcore_barrier()


def _spmm_body(tbl_hbm, src_hbm, dst_hbm, ones_hbm, agg_hbm,
               acc, srcb, dstb, rows, zed, gsems, ssems, isem):
    cid = lax.axis_index("c")
    tid = lax.axis_index("s")
    pltpu.sync_copy(ones_hbm.at[pl.ds(CHUNK, ZROWS)], zed)
    for r in range(3):
        _relation_pass(tbl_hbm, src_hbm, dst_hbm, agg_hbm, acc, srcb, dstb,
                       rows, zed, gsems, ssems, isem, tid, cid,
                       r, cid, 0, NGRP, 2 * HALF * r + HALF * cid)


def _head_body(tbl_hbm, src_hbm, dst_hbm, ones_hbm, agg_hbm, cnt_hbm,
               acc, srcb, dstb, rows, zed, gsems, ssems, isem):
    cid = lax.axis_index("c")
    tid = lax.axis_index("s")
    pltpu.sync_copy(ones_hbm.at[pl.ds(CHUNK, ZROWS)], zed)
    for r in range(3):
        _relation_pass(tbl_hbm, src_hbm, dst_hbm, agg_hbm, acc, srcb, dstb,
                       rows, zed, gsems, ssems, isem, tid, cid,
                       r, 0, cid * NCH2, NGRP2, 2 * HALF * r + HALF * cid)

    rel = cid * 2

    @pl.loop(0, NZC)
    def _zero(k):
        pltpu.sync_copy(zed, acc.at[pl.ds(tid * STRIPE + k * ZROWS, ZROWS)])

    pltpu.sync_copy(ones_hbm.at[pl.ds(0, CHUNK)], rows.at[0])
    plsc.subcore_barrier()

    @pl.loop(0, NGRP)
    def _grp(g):
        base = tid * CPT + g * GRP
        pltpu.sync_copy(dst_hbm.at[rel, pl.ds(base, GRP)], dstb.at[0])
        cps = [pltpu.async_copy(rows.at[0], acc.at[dstb.at[0, j]],
                                ssems.at[j], add=True)
               for j in range(GRP)]
        for cp in cps:
            cp.wait()

    plsc.subcore_barrier()
    pltpu.sync_copy(acc.at[pl.ds(tid * STRIPE, STRIPE)],
                    cnt_hbm.at[cid, pl.ds(tid * STRIPE, STRIPE)])


_head_call = pl.kernel(
    _head_body,
    out_type=[jax.ShapeDtypeStruct((NP, 8 * HALF), jnp.float32),
              jax.ShapeDtypeStruct((2, NP, HALF), jnp.float32)],
    mesh=_mesh,
    compiler_params=pltpu.CompilerParams(use_tc_tiling_on_sc=False),
    scratch_types=[
        pltpu.VMEM_SHARED((NP, HALF), jnp.float32),
        pltpu.VMEM((2, GRP, CHUNK), jnp.int32),
        pltpu.VMEM((2, GRP, CHUNK), jnp.int32),
        pltpu.VMEM((GRP, CHUNK, HALF), jnp.float32),
        pltpu.VMEM((ZROWS, HALF), jnp.float32),
        pltpu.SemaphoreType.DMA((GRP,)),
        pltpu.SemaphoreType.DMA((GRP,)),
        pltpu.SemaphoreType.DMA,
    ],
    name="sc_head",
)

_spmm_call = pl.kernel(
    _spmm_body,
    out_type=jax.ShapeDtypeStruct((NP, 8 * HALF), jnp.float32),
    mesh=_mesh,
    compiler_params=pltpu.CompilerParams(use_tc_tiling_on_sc=False),
    scratch_types=[
        pltpu.VMEM_SHARED((NP, HALF), jnp.float32),
        pltpu.VMEM((2, GRP, CHUNK), jnp.int32),
        pltpu.VMEM((2, GRP, CHUNK), jnp.int32),
        pltpu.VMEM((GRP, CHUNK, HALF), jnp.float32),
        pltpu.VMEM((ZROWS, HALF), jnp.float32),
        pltpu.SemaphoreType.DMA((GRP,)),
        pltpu.SemaphoreType.DMA((GRP,)),
        pltpu.SemaphoreType.DMA,
    ],
    name="sc_spmm",
)


def _tc_head_body(agg_ref, h_ref, cnt_ref, m_ref, w_ref, b_ref,
                  out_ref, scale_ref):
    inv0 = 1.0 / jnp.maximum(cnt_ref[0], 1.0)
    inv2 = 1.0 / jnp.maximum(cnt_ref[1], 1.0)
    one = jnp.ones_like(inv0)
    scale_ref[...] = jnp.concatenate([inv0, inv0, one, one, inv2, inv2],
                                     axis=-1)
    a = agg_ref[...]
    a0 = (a[:, 0:HALF] + a[:, HALF:2 * HALF]) * inv0
    a1 = a[:, 2 * HALF:3 * HALF] + a[:, 3 * HALF:4 * HALF]
    a2 = (a[:, 4 * HALF:5 * HALF] + a[:, 5 * HALF:6 * HALF]) * inv2
    za = jnp.concatenate([a0, a1, a2], axis=-1)
    h = h_ref[...]
    out_ref[...] = (jnp.dot(za, m_ref[...], preferred_element_type=jnp.float32)
                    + jnp.dot(h, w_ref[...],
                              preferred_element_type=jnp.float32)
                    + b_ref[...])


def _tc_body(agg_ref, h_ref, scale_ref, m_ref, w_ref, b_ref, out_ref, *,
             mode):
    za = agg_ref[:, :6 * HALF] * scale_ref[...]
    h = h_ref[...]
    o = (jnp.dot(za, m_ref[...], preferred_element_type=jnp.float32)
         + jnp.dot(h, w_ref[...], preferred_element_type=jnp.float32)
         + b_ref[...])
    hn = h + jnp.maximum(o, 0.0)
    if mode == "final":
        hn = jnp.maximum(hn, 0.0)
    out_ref[...] = hn


_TCB = 2176


def _tc_head_call(agg, h, cnt, m, w, bias):
    grid = (NP // _TCB,)
    in_specs = [
        pl.BlockSpec((_TCB, 8 * HALF), lambda i: (i, 0)),
        pl.BlockSpec((_TCB, HID), lambda i: (i, 0)),
        pl.BlockSpec((2, _TCB, HALF), lambda i: (0, i, 0)),
        pl.BlockSpec((3 * HALF, HID), lambda i: (0, 0)),
        pl.BlockSpec((HID, HID), lambda i: (0, 0)),
        pl.BlockSpec((1, HID), lambda i: (0, 0)),
    ]
    return pl.pallas_call(
        _tc_head_body,
        grid=grid,
        in_specs=in_specs,
        out_specs=[pl.BlockSpec((_TCB, HID), lambda i: (i, 0)),
                   pl.BlockSpec((_TCB, 6 * HALF), lambda i: (i, 0))],
        out_shape=[jax.ShapeDtypeStruct((NP, HID), jnp.float32),
                   jax.ShapeDtypeStruct((NP, 6 * HALF), jnp.float32)],
        name="tc_dense_head",
    )(agg, h, cnt, m, w, bias)


def _tc_call(agg, h, scale, m, w, bias, mode):
    grid = (NP // _TCB,)
    in_specs = [
        pl.BlockSpec((_TCB, 8 * HALF), lambda i: (i, 0)),
        pl.BlockSpec((_TCB, HID), lambda i: (i, 0)),
        pl.BlockSpec((_TCB, 6 * HALF), lambda i: (i, 0)),
        pl.BlockSpec((6 * HALF, HID), lambda i: (0, 0)),
        pl.BlockSpec((HID, HID), lambda i: (0, 0)),
        pl.BlockSpec((1, HID), lambda i: (0, 0)),
    ]
    return pl.pallas_call(
        functools.partial(_tc_body, mode=mode),
        grid=grid,
        in_specs=in_specs,
        out_specs=pl.BlockSpec((_TCB, HID), lambda i: (i, 0)),
        out_shape=jax.ShapeDtypeStruct((NP, HID), jnp.float32),
        name=f"tc_dense_{mode}",
    )(agg, h, scale, m, w, bias)


def _prep_edges(ei):
    src = ei[0].astype(jnp.int32)
    dst = ei[1].astype(jnp.int32)
    src = jnp.concatenate([src, jnp.zeros((PAD,), jnp.int32)])
    dst = jnp.concatenate([dst, jnp.full((PAD,), DUMP, jnp.int32)])
    src2 = jnp.stack([2 * src, 2 * src + 1]).reshape(2, NCH, CHUNK)
    return src2, dst.reshape(NCH, CHUNK)


def _pad_rows(w):
    return jnp.pad(w, ((0, HID - w.shape[0]), (0, 0)))


def _pad16(w):
    return jnp.pad(w, ((0, HALF - w.shape[0]), (0, 0)))


def kernel(x_stroke, edge_index_intersects, edge_index_temp_previous,
           edge_index_represented_by, W_msg_head, W_self_head, b_head,
           W_msg, W_self, b):
    f32 = jnp.float32
    prepped = [_prep_edges(e) for e in (edge_index_intersects,
                                        edge_index_temp_previous,
                                        edge_index_represented_by)]
    src_all = jnp.stack([p[0] for p in prepped])
    dst_all = jnp.stack([p[1] for p in prepped])

    ones_z = jnp.concatenate([jnp.ones((CHUNK, HALF), f32),
                              jnp.zeros((ZROWS, HALF), f32)])

    m_head = jnp.concatenate([_pad16(W_msg_head[0].astype(f32)),
                              _pad16(W_msg_head[1].astype(f32)),
                              _pad16(W_msg_head[2].astype(f32))])
    w_head = _pad_rows(W_self_head.astype(f32).sum(0))
    b_hd = b_head.astype(f32).sum(0).reshape(1, HID)
    ms = [jnp.concatenate([W_msg[l, 0], W_msg[l, 1],
                           W_msg[l, 2]]).astype(f32) for l in range(4)]
    ws = [W_self[l].sum(0).astype(f32) for l in range(4)]
    bs = [b[l].astype(f32).sum(0).reshape(1, HID) for l in range(4)]

    h = jnp.pad(x_stroke.astype(f32),
                ((0, NP - N), (0, HID - x_stroke.shape[1])))

    agg, cnt = _head_call(h.reshape(2 * NP, HALF), src_all, dst_all, ones_z)
    h, scale = _tc_head_call(agg, h, cnt, m_head, w_head, b_hd)
    for l in range(4):
        agg = _spmm_call(h.reshape(2 * NP, HALF), src_all, dst_all, ones_z)
        mode = "final" if l == 3 else "mid"
        h = _tc_call(agg, h, scale, ms[l], ws[l], bs[l], mode)
    return h[:N]
